# baseline (device time: 224721 ns/iter reference)
import jax
import jax.numpy as jnp
from jax import lax
from jax.experimental import pallas as pl
from jax.experimental.pallas import tpu as pltpu

K = 16


def kernel(x):
    m, n = x.shape
    h = m // 2
    r = h // K

    def body(x_ref, out_ref, ry_ref, rx_ref, xs, rs, os,
             copy_sems, sy_sems, ry_sems, sx_sems, rx_sems):
        a = lax.axis_index("x")
        b = lax.axis_index("y")
        y_nbr = (a, 1 - b)
        x_nbr = (1 - a, b)

        barrier_sem = pltpu.get_barrier_semaphore()
        for nbr in (y_nbr, x_nbr):
            pl.semaphore_signal(
                barrier_sem, inc=1, device_id=nbr,
                device_id_type=pl.DeviceIdType.MESH,
            )
        pl.semaphore_wait(barrier_sem, 2)

        my_half = a * h
        other_half = (1 - a) * h

        def add_chunk(recv_ref, ck, rows):
            cx = pltpu.make_async_copy(x_ref.at[rows], xs, copy_sems.at[0])
            cr = pltpu.make_async_copy(recv_ref.at[ck], rs, copy_sems.at[1])
            cx.start()
            cr.start()
            cx.wait()
            cr.wait()
            os[:, :] = xs[:, :] + rs[:, :]
            co = pltpu.make_async_copy(os, out_ref.at[rows], copy_sems.at[2])
            co.start()
            co.wait()

        def wait_recv_y(k):
            ck = pl.ds(k * r, r)
            pltpu.make_async_remote_copy(
                src_ref=x_ref.at[ck],
                dst_ref=ry_ref.at[ck],
                send_sem=sy_sems.at[k],
                recv_sem=ry_sems.at[k],
                device_id=y_nbr,
                device_id_type=pl.DeviceIdType.MESH,
            ).wait_recv()

        def wait_recv_x(k):
            ck = pl.ds(k * r, r)
            pltpu.make_async_remote_copy(
                src_ref=ry_ref.at[ck],
                dst_ref=rx_ref.at[ck],
                send_sem=sx_sems.at[k],
                recv_sem=rx_sems.at[k],
                device_id=x_nbr,
                device_id_type=pl.DeviceIdType.MESH,
            ).wait_recv()

        for k in range(K):
            pltpu.make_async_remote_copy(
                src_ref=x_ref.at[pl.ds(my_half + k * r, r)],
                dst_ref=ry_ref.at[pl.ds(k * r, r)],
                send_sem=sy_sems.at[k],
                recv_sem=ry_sems.at[k],
                device_id=y_nbr,
                device_id_type=pl.DeviceIdType.MESH,
            ).start()

        for k in range(K):
            ck = pl.ds(k * r, r)
            wait_recv_y(k)
            pltpu.make_async_remote_copy(
                src_ref=ry_ref.at[ck],
                dst_ref=rx_ref.at[ck],
                send_sem=sx_sems.at[k],
                recv_sem=rx_sems.at[k],
                device_id=x_nbr,
                device_id_type=pl.DeviceIdType.MESH,
            ).start()
            add_chunk(ry_ref, ck, pl.ds(my_half + k * r, r))
            if k >= 1:
                j = k - 1
                wait_recv_x(j)
                add_chunk(rx_ref, pl.ds(j * r, r), pl.ds(other_half + j * r, r))

        wait_recv_x(K - 1)
        add_chunk(
            rx_ref, pl.ds((K - 1) * r, r), pl.ds(other_half + (K - 1) * r, r)
        )

        for k in range(K):
            pltpu.make_async_remote_copy(
                src_ref=x_ref.at[pl.ds(my_half + k * r, r)],
                dst_ref=ry_ref.at[pl.ds(k * r, r)],
                send_sem=sy_sems.at[k],
                recv_sem=ry_sems.at[k],
                device_id=y_nbr,
                device_id_type=pl.DeviceIdType.MESH,
            ).wait_send()
            pltpu.make_async_remote_copy(
                src_ref=ry_ref.at[pl.ds(k * r, r)],
                dst_ref=rx_ref.at[pl.ds(k * r, r)],
                send_sem=sx_sems.at[k],
                recv_sem=rx_sems.at[k],
                device_id=x_nbr,
                device_id_type=pl.DeviceIdType.MESH,
            ).wait_send()

    out, _, _ = pl.pallas_call(
        body,
        out_shape=[
            jax.ShapeDtypeStruct((m, n), x.dtype),
            jax.ShapeDtypeStruct((h, n), x.dtype),
            jax.ShapeDtypeStruct((h, n), x.dtype),
        ],
        in_specs=[pl.BlockSpec(memory_space=pltpu.HBM)],
        out_specs=[
            pl.BlockSpec(memory_space=pltpu.HBM),
            pl.BlockSpec(memory_space=pltpu.HBM),
            pl.BlockSpec(memory_space=pltpu.HBM),
        ],
        scratch_shapes=[
            pltpu.VMEM((r, n), x.dtype),
            pltpu.VMEM((r, n), x.dtype),
            pltpu.VMEM((r, n), x.dtype),
            pltpu.SemaphoreType.DMA((3,)),
            pltpu.SemaphoreType.DMA((K,)),
            pltpu.SemaphoreType.DMA((K,)),
            pltpu.SemaphoreType.DMA((K,)),
            pltpu.SemaphoreType.DMA((K,)),
        ],
        compiler_params=pltpu.CompilerParams(collective_id=0),
    )(x)
    return out


# device time: 218921 ns/iter; 1.0265x vs baseline; 1.0265x over previous
import jax
import jax.numpy as jnp
from jax import lax
from jax.experimental import pallas as pl
from jax.experimental.pallas import tpu as pltpu

K = 32


def kernel(x):
    m, n = x.shape
    h = m // 2
    r = h // K

    def body(x_ref, out_ref, ry_ref, rx_ref, xs, rs, os,
             copy_sems, sy_sems, ry_sems, sx_sems, rx_sems):
        a = lax.axis_index("x")
        b = lax.axis_index("y")
        y_nbr = (a, 1 - b)
        x_nbr = (1 - a, b)

        barrier_sem = pltpu.get_barrier_semaphore()
        for nbr in (y_nbr, x_nbr):
            pl.semaphore_signal(
                barrier_sem, inc=1, device_id=nbr,
                device_id_type=pl.DeviceIdType.MESH,
            )
        pl.semaphore_wait(barrier_sem, 2)

        my_half = a * h
        other_half = (1 - a) * h

        def add_chunk(recv_ref, ck, rows):
            cx = pltpu.make_async_copy(x_ref.at[rows], xs, copy_sems.at[0])
            cr = pltpu.make_async_copy(recv_ref.at[ck], rs, copy_sems.at[1])
            cx.start()
            cr.start()
            cx.wait()
            cr.wait()
            os[:, :] = xs[:, :] + rs[:, :]
            co = pltpu.make_async_copy(os, out_ref.at[rows], copy_sems.at[2])
            co.start()
            co.wait()

        def wait_recv_y(k):
            ck = pl.ds(k * r, r)
            pltpu.make_async_remote_copy(
                src_ref=x_ref.at[ck],
                dst_ref=ry_ref.at[ck],
                send_sem=sy_sems.at[k],
                recv_sem=ry_sems.at[k],
                device_id=y_nbr,
                device_id_type=pl.DeviceIdType.MESH,
            ).wait_recv()

        def wait_recv_x(k):
            ck = pl.ds(k * r, r)
            pltpu.make_async_remote_copy(
                src_ref=ry_ref.at[ck],
                dst_ref=rx_ref.at[ck],
                send_sem=sx_sems.at[k],
                recv_sem=rx_sems.at[k],
                device_id=x_nbr,
                device_id_type=pl.DeviceIdType.MESH,
            ).wait_recv()

        for k in range(K):
            pltpu.make_async_remote_copy(
                src_ref=x_ref.at[pl.ds(my_half + k * r, r)],
                dst_ref=ry_ref.at[pl.ds(k * r, r)],
                send_sem=sy_sems.at[k],
                recv_sem=ry_sems.at[k],
                device_id=y_nbr,
                device_id_type=pl.DeviceIdType.MESH,
            ).start()

        for k in range(K):
            ck = pl.ds(k * r, r)
            wait_recv_y(k)
            pltpu.make_async_remote_copy(
                src_ref=ry_ref.at[ck],
                dst_ref=rx_ref.at[ck],
                send_sem=sx_sems.at[k],
                recv_sem=rx_sems.at[k],
                device_id=x_nbr,
                device_id_type=pl.DeviceIdType.MESH,
            ).start()
            add_chunk(ry_ref, ck, pl.ds(my_half + k * r, r))
            if k >= 1:
                j = k - 1
                wait_recv_x(j)
                add_chunk(rx_ref, pl.ds(j * r, r), pl.ds(other_half + j * r, r))

        wait_recv_x(K - 1)
        add_chunk(
            rx_ref, pl.ds((K - 1) * r, r), pl.ds(other_half + (K - 1) * r, r)
        )

        for k in range(K):
            pltpu.make_async_remote_copy(
                src_ref=x_ref.at[pl.ds(my_half + k * r, r)],
                dst_ref=ry_ref.at[pl.ds(k * r, r)],
                send_sem=sy_sems.at[k],
                recv_sem=ry_sems.at[k],
                device_id=y_nbr,
                device_id_type=pl.DeviceIdType.MESH,
            ).wait_send()
            pltpu.make_async_remote_copy(
                src_ref=ry_ref.at[pl.ds(k * r, r)],
                dst_ref=rx_ref.at[pl.ds(k * r, r)],
                send_sem=sx_sems.at[k],
                recv_sem=rx_sems.at[k],
                device_id=x_nbr,
                device_id_type=pl.DeviceIdType.MESH,
            ).wait_send()

    out, _, _ = pl.pallas_call(
        body,
        out_shape=[
            jax.ShapeDtypeStruct((m, n), x.dtype),
            jax.ShapeDtypeStruct((h, n), x.dtype),
            jax.ShapeDtypeStruct((h, n), x.dtype),
        ],
        in_specs=[pl.BlockSpec(memory_space=pltpu.HBM)],
        out_specs=[
            pl.BlockSpec(memory_space=pltpu.HBM),
            pl.BlockSpec(memory_space=pltpu.HBM),
            pl.BlockSpec(memory_space=pltpu.HBM),
        ],
        scratch_shapes=[
            pltpu.VMEM((r, n), x.dtype),
            pltpu.VMEM((r, n), x.dtype),
            pltpu.VMEM((r, n), x.dtype),
            pltpu.SemaphoreType.DMA((3,)),
            pltpu.SemaphoreType.DMA((K,)),
            pltpu.SemaphoreType.DMA((K,)),
            pltpu.SemaphoreType.DMA((K,)),
            pltpu.SemaphoreType.DMA((K,)),
        ],
        compiler_params=pltpu.CompilerParams(collective_id=0),
    )(x)
    return out
